# baseline (speedup 1.0000x reference)
import functools

import jax
import jax.numpy as jnp
from jax import lax
from jax.experimental import pallas as pl
from jax.experimental.pallas import tpu as pltpu
from jax.experimental.pallas import tpu_sc as plsc

_CW = 128
_NW = 32
_ROWS = 4096
_RPW = _ROWS // _NW

_BR = 256


def _sc_compact_last(x_hbm, col_hbm, gath_v, sem):
    L = x_hbm.shape[1]
    wid = lax.axis_index("s") * 2 + lax.axis_index("c")
    base = wid * _RPW
    pltpu.async_copy(
        x_hbm.at[pl.ds(base, _RPW), pl.ds(L - _CW, _CW)], gath_v, sem,
    ).wait()
    pltpu.sync_copy(gath_v, col_hbm.at[pl.ds(base, _RPW), :])


def _tc_fill_kernel(o_ref):
    o_ref[...] = jnp.zeros(o_ref.shape, o_ref.dtype)


def _tc_merge_kernel(zero_ref, col_ref, o_ref):
    del zero_ref
    cid = lax.broadcasted_iota(jnp.int32, o_ref.shape, 1)
    o_ref[...] = jnp.where(cid == 1, col_ref[:, _CW - 1:_CW], 0.0)


def kernel(x):
    N, C, L = x.shape
    L_out = L + 1
    rows = N * C
    x2 = x.reshape(rows, L)

    mesh = plsc.VectorSubcoreMesh(core_axis_name="c", subcore_axis_name="s")
    sc_compact = functools.partial(
        pl.kernel,
        mesh=mesh,
        out_type=jax.ShapeDtypeStruct((rows, _CW), jnp.float32),
        scratch_types=[
            pltpu.VMEM((_RPW, _CW), jnp.float32),
            pltpu.SemaphoreType.DMA,
        ],
    )(_sc_compact_last)
    col = sc_compact(x2)

    zeros2 = pl.pallas_call(
        _tc_fill_kernel,
        grid=(rows // _BR,),
        out_specs=pl.BlockSpec((_BR, L_out), lambda i: (i, 0)),
        out_shape=jax.ShapeDtypeStruct((rows, L_out), x.dtype),
    )()

    out2 = pl.pallas_call(
        _tc_merge_kernel,
        grid=(rows // _BR,),
        in_specs=[
            pl.BlockSpec((_BR, _CW), lambda i: (i, 0)),
            pl.BlockSpec((_BR, _CW), lambda i: (i, 0)),
        ],
        out_specs=pl.BlockSpec((_BR, _CW), lambda i: (i, 0)),
        out_shape=jax.ShapeDtypeStruct((rows, L_out), x.dtype),
        input_output_aliases={0: 0},
    )(zeros2, col)
    return out2.reshape(N, C, L_out)


# speedup vs baseline: 1.0312x; 1.0312x over previous
import functools

import jax
import jax.numpy as jnp
from jax import lax
from jax.experimental import pallas as pl
from jax.experimental.pallas import tpu as pltpu
from jax.experimental.pallas import tpu_sc as plsc

_CW = 128
_NW = 32
_ROWS = 4096
_RPW = _ROWS // _NW

_BR = 256


def _sc_compact_last(x_hbm, col_hbm, gath_v, sem):
    L = x_hbm.shape[1]
    wid = lax.axis_index("s") * 2 + lax.axis_index("c")
    base = wid * _RPW
    pltpu.async_copy(
        x_hbm.at[pl.ds(base, _RPW), pl.ds(L - _CW, _CW)], gath_v, sem,
    ).wait()
    pltpu.sync_copy(gath_v, col_hbm.at[pl.ds(base, _RPW), :])


def _tc_fill_kernel(col_ref, o_ref):
    o_ref[...] = jnp.zeros(o_ref.shape, o_ref.dtype)
    cid = lax.broadcasted_iota(jnp.int32, (o_ref.shape[0], _CW), 1)
    o_ref[:, 0:_CW] = jnp.where(cid == 1, col_ref[:, _CW - 1:_CW], 0.0)


def kernel(x):
    N, C, L = x.shape
    L_out = L + 1
    rows = N * C
    x2 = x.reshape(rows, L)

    mesh = plsc.VectorSubcoreMesh(core_axis_name="c", subcore_axis_name="s")
    sc_compact = functools.partial(
        pl.kernel,
        mesh=mesh,
        out_type=jax.ShapeDtypeStruct((rows, _CW), jnp.float32),
        scratch_types=[
            pltpu.VMEM((_RPW, _CW), jnp.float32),
            pltpu.SemaphoreType.DMA,
        ],
    )(_sc_compact_last)
    col = sc_compact(x2)

    out2 = pl.pallas_call(
        _tc_fill_kernel,
        grid=(rows // _BR,),
        in_specs=[pl.BlockSpec((_BR, _CW), lambda i: (i, 0))],
        out_specs=pl.BlockSpec((_BR, L_out), lambda i: (i, 0)),
        out_shape=jax.ShapeDtypeStruct((rows, L_out), x.dtype),
    )(col)
    return out2.reshape(N, C, L_out)
